# Initial kernel scaffold; baseline (speedup 1.0000x reference)
import jax
import jax.numpy as jnp
from jax.experimental import pallas as pl


def kernel(x, codebook):
    raise NotImplementedError("write your pallas kernel here")


# speedup vs baseline: 2.3526x; 2.3526x over previous
import jax
import jax.numpy as jnp
from jax.experimental import pallas as pl

_NE = 512
_ED = 64
_BN = 256


def _qkernel(x_ref, cb_ref, cbt_ref, out_ref, ssq_ref):
    i = pl.program_id(0)
    xb = x_ref[...]
    cb = cb_ref[...]
    cbt = cbt_ref[...]

    acc = None
    for g in range(8):
        d2 = []
        for p in range(8):
            j = g * 8 + p
            t = xb[:, j:j + 1] - cbt[j:j + 1, :]
            d2.append(t * t)
        v = [d2[p] + d2[p + 4] for p in range(4)]
        w = [v[p] + v[p + 2] for p in range(2)]
        f = w[0] + w[1]
        acc = f if acc is None else acc + f

    dmin = jnp.min(acc, axis=1, keepdims=True)
    kiota = jax.lax.broadcasted_iota(jnp.int32, acc.shape, 1)
    idx = jnp.min(jnp.where(acc == dmin, kiota, _NE), axis=1)
    oh = (kiota == idx[:, None]).astype(jnp.float32)
    q = jax.lax.dot_general(oh, cb, (((1,), (0,)), ((), ())),
                            precision=jax.lax.Precision.HIGHEST,
                            preferred_element_type=jnp.float32)
    out_ref[...] = xb + (q - xb)
    part = jnp.sum((q - xb) ** 2).reshape(1, 1)
    ssq_ref[...] = jnp.where(i == 0, part, ssq_ref[...] + part)


def kernel(x, codebook):
    shape = x.shape
    xf = x.reshape(-1, _ED)
    n = xf.shape[0]
    out, ssq = pl.pallas_call(
        _qkernel,
        grid=(n // _BN,),
        in_specs=[pl.BlockSpec((_BN, _ED), lambda i: (i, 0)),
                  pl.BlockSpec((_NE, _ED), lambda i: (0, 0)),
                  pl.BlockSpec((_ED, _NE), lambda i: (0, 0))],
        out_specs=[pl.BlockSpec((_BN, _ED), lambda i: (i, 0)),
                   pl.BlockSpec((1, 1), lambda i: (0, 0))],
        out_shape=[jax.ShapeDtypeStruct((n, _ED), jnp.float32),
                   jax.ShapeDtypeStruct((1, 1), jnp.float32)],
    )(xf, codebook, codebook.T)
    m = ssq[0, 0] / jnp.float32(n * _ED)
    loss = m + jnp.float32(0.25) * m
    return out.reshape(shape), loss
